# baseline (speedup 1.0000x reference)
import functools

import jax
import jax.numpy as jnp
from jax import lax
from jax.experimental import pallas as pl
from jax.experimental.pallas import tpu as pltpu
from jax.experimental.pallas import tpu_sc as plsc

NB_CLASSES = 1000000
NB_RELS = 1000
D = 32
B = 16384
GAMMA = 10.0

NC, NS, L = 2, 16, 16
NW = NC * NS
BPW = B // NW
CHUNK = 128
NCHUNK = BPW // CHUNK
NGROUP = BPW // L


def _sqrt16(x):
    xs = jnp.maximum(x, jnp.float32(1e-30))
    i = lax.bitcast_convert_type(xs, jnp.int32)
    i = jnp.int32(0x5F3759DF) - lax.shift_right_logical(i, 1)
    y = lax.bitcast_convert_type(i, jnp.float32)
    half = jnp.float32(0.5) * xs
    for _ in range(3):
        y = y * (jnp.float32(1.5) - half * y * y)
    return x * y


def _body(heads_hbm, rels_hbm, tails_hbm, ce_hbm, co_hbm, rel_hbm, out_hbm,
          hv, rv, tv, ce_v, co_v, ans_v, rel_v, out_v, sem):
    cid = lax.axis_index("c")
    sid = lax.axis_index("s")
    wid = sid * NC + cid
    base4 = wid * NCHUNK

    pltpu.sync_copy(heads_hbm.at[pl.ds(base4, NCHUNK)], hv)
    pltpu.sync_copy(rels_hbm.at[pl.ds(base4, NCHUNK)], rv)
    pltpu.sync_copy(tails_hbm.at[pl.ds(base4, NCHUNK)], tv)

    copies = []
    for j in range(NCHUNK):
        dst = pl.ds(j * CHUNK, CHUNK)
        copies.append(pltpu.async_copy(ce_hbm.at[hv.at[j]], ce_v.at[dst], sem))
        copies.append(pltpu.async_copy(co_hbm.at[hv.at[j]], co_v.at[dst], sem))
        copies.append(pltpu.async_copy(ce_hbm.at[tv.at[j]], ans_v.at[dst], sem))
        copies.append(pltpu.async_copy(rel_hbm.at[rv.at[j]], rel_v.at[dst], sem))
    for cp in copies:
        cp.wait()

    lanes = lax.iota(jnp.int32, L)

    def row_sums(r):
        so = jnp.zeros((L,), jnp.float32)
        si = jnp.zeros((L,), jnp.float32)
        for h in range(D // L):
            sl = pl.ds(h * L, L)
            cc = ce_v[r, sl]
            oo = co_v[r, sl]
            aa = ans_v[r, sl]
            rt = rel_v[r, pl.ds(h * L, L)]
            rf = rel_v[r, pl.ds(D + h * L, L)]
            rs = rel_v[r, pl.ds(2 * D + h * L, L)]
            rb = rel_v[r, pl.ds(3 * D + h * L, L)]
            cc = cc * rf + rt
            off = jnp.abs(oo) * jnp.abs(rs) + jnp.abs(rb)
            delta = jnp.abs(cc - aa)
            dout = jnp.maximum(delta - off, jnp.float32(0.0))
            din = jnp.minimum(delta, off)
            so = so + dout * dout
            si = si + din * din
        return jnp.sum(so), jnp.sum(si)

    def group(g, carry):
        acc_o = jnp.zeros((L,), jnp.float32)
        acc_i = jnp.zeros((L,), jnp.float32)
        for k in range(L):
            sum_o, sum_i = row_sums(g * L + k)
            m = lanes == k
            acc_o = jnp.where(m, sum_o, acc_o)
            acc_i = jnp.where(m, sum_i, acc_i)
        dist = _sqrt16(acc_o) + jnp.float32(0.5) * _sqrt16(acc_i)
        out_v[pl.ds(g * L, L)] = jnp.float32(GAMMA) - dist
        return carry

    lax.fori_loop(0, NGROUP, group, 0)

    pltpu.sync_copy(out_v, out_hbm.at[pl.ds(wid * BPW, BPW)])


@functools.partial(jax.jit, static_argnames=())
def _run(heads2, rels2, tails2, class_embed, class_offset, rel_all):
    mesh = plsc.VectorSubcoreMesh(core_axis_name="c", subcore_axis_name="s")
    k = functools.partial(
        pl.kernel,
        out_type=jax.ShapeDtypeStruct((B,), jnp.float32),
        mesh=mesh,
        compiler_params=pltpu.CompilerParams(
            needs_layout_passes=False, use_tc_tiling_on_sc=False),
        scratch_types=[
            pltpu.VMEM((NCHUNK, CHUNK), jnp.int32),
            pltpu.VMEM((NCHUNK, CHUNK), jnp.int32),
            pltpu.VMEM((NCHUNK, CHUNK), jnp.int32),
            pltpu.VMEM((BPW, D), jnp.float32),
            pltpu.VMEM((BPW, D), jnp.float32),
            pltpu.VMEM((BPW, D), jnp.float32),
            pltpu.VMEM((BPW, 4 * D), jnp.float32),
            pltpu.VMEM((BPW,), jnp.float32),
            pltpu.SemaphoreType.DMA,
        ],
    )(_body)
    return k(heads2, rels2, tails2, class_embed, class_offset, rel_all)


def kernel(heads, rels, tails, class_embed, class_offset, rel_embed,
           rel_factor, scale_embed, scale_bias):
    rel_all = jnp.concatenate(
        [rel_embed, rel_factor, scale_embed, scale_bias], axis=1)
    heads2 = heads.astype(jnp.int32).reshape(NW * NCHUNK, CHUNK)
    rels2 = rels.astype(jnp.int32).reshape(NW * NCHUNK, CHUNK)
    tails2 = tails.astype(jnp.int32).reshape(NW * NCHUNK, CHUNK)
    return _run(heads2, rels2, tails2, class_embed, class_offset, rel_all)


# speedup vs baseline: 1.3564x; 1.3564x over previous
import functools

import jax
import jax.numpy as jnp
from jax import lax
from jax.experimental import pallas as pl
from jax.experimental.pallas import tpu as pltpu
from jax.experimental.pallas import tpu_sc as plsc

NB_CLASSES = 1000000
NB_RELS = 1000
D = 32
B = 16384
GAMMA = 10.0

NC, NS, L = 2, 16, 16
NW = NC * NS
BPW = B // NW
CHUNK = 128
NCHUNK = BPW // CHUNK
GPC = CHUNK // L

RB = 2048
RGRID = (NB_CLASSES + RB - 1) // RB
RROWS = RGRID * (RB // 4)


def _repack_body(ce_ref, co_ref, cep_ref, cop_ref):
    et = jnp.transpose(ce_ref[...])
    ot = jnp.transpose(co_ref[...])
    q = RB // 4
    for a in range(4):
        cep_ref[:, a * D:(a + 1) * D] = et[a * q:(a + 1) * q, :]
        cop_ref[:, a * D:(a + 1) * D] = ot[a * q:(a + 1) * q, :]


def _repack(ceT, coT):
    return pl.pallas_call(
        _repack_body,
        grid=(RGRID,),
        in_specs=[pl.BlockSpec((D, RB), lambda i: (0, i)),
                  pl.BlockSpec((D, RB), lambda i: (0, i))],
        out_specs=[pl.BlockSpec((RB // 4, 4 * D), lambda i: (i, 0)),
                   pl.BlockSpec((RB // 4, 4 * D), lambda i: (i, 0))],
        out_shape=[jax.ShapeDtypeStruct((RROWS, 4 * D), jnp.float32),
                   jax.ShapeDtypeStruct((RROWS, 4 * D), jnp.float32)],
    )(ceT, coT)


def _sqrt16(x):
    xs = jnp.maximum(x, jnp.float32(1e-30))
    i = lax.bitcast_convert_type(xs, jnp.int32)
    i = jnp.int32(0x5F3759DF) - lax.shift_right_logical(i, 1)
    y = lax.bitcast_convert_type(i, jnp.float32)
    half = jnp.float32(0.5) * xs
    for _ in range(3):
        y = y * (jnp.float32(1.5) - half * y * y)
    return x * y


def _body(heads_hbm, rels_hbm, tails_hbm, cep_hbm, cop_hbm, rel_hbm, out_hbm,
          hv, rv, tv, hrow, trow, hc_buf, ho_buf, tc_buf, r_buf, out_v, sem):
    cid = lax.axis_index("c")
    sid = lax.axis_index("s")
    wid = sid * NC + cid
    base = wid * BPW

    pltpu.sync_copy(heads_hbm.at[pl.ds(base, BPW)], hv)
    pltpu.sync_copy(rels_hbm.at[pl.ds(base, BPW)], rv)
    pltpu.sync_copy(tails_hbm.at[pl.ds(base, BPW)], tv)

    def rowify(k, carry):
        sl = pl.ds(k * L, L)
        h = hv[sl]
        t = tv[sl]
        hrow[sl] = lax.shift_left(lax.shift_right_logical(h, 11), 9) + \
            jnp.bitwise_and(h, 511)
        trow[sl] = lax.shift_left(lax.shift_right_logical(t, 11), 9) + \
            jnp.bitwise_and(t, 511)
        return carry
    lax.fori_loop(0, BPW // L, rowify, 0)

    lanes = lax.iota(jnp.int32, L)

    def chunk(j, carry):
        isl = pl.ds(j * CHUNK, CHUNK)
        cp1 = pltpu.async_copy(cep_hbm.at[hrow.at[isl]], hc_buf, sem)
        cp2 = pltpu.async_copy(cop_hbm.at[hrow.at[isl]], ho_buf, sem)
        cp3 = pltpu.async_copy(cep_hbm.at[trow.at[isl]], tc_buf, sem)
        cp4 = pltpu.async_copy(rel_hbm.at[rv.at[isl]], r_buf, sem)
        cp1.wait()
        cp2.wait()
        cp3.wait()
        cp4.wait()

        def group(g, gcarry):
            rows = g * L + lanes
            sl = pl.ds(j * CHUNK + g * L, L)
            h = hv[sl]
            t = tv[sl]
            hq = lax.shift_left(
                jnp.bitwise_and(lax.shift_right_logical(h, 9), 3), 5)
            tq = lax.shift_left(
                jnp.bitwise_and(lax.shift_right_logical(t, 9), 3), 5)
            acc_o = jnp.zeros((L,), jnp.float32)
            acc_i = jnp.zeros((L,), jnp.float32)
            for d in range(D):
                cc = plsc.load_gather(hc_buf, [rows, hq + d])
                oo = plsc.load_gather(ho_buf, [rows, hq + d])
                aa = plsc.load_gather(tc_buf, [rows, tq + d])
                rt = plsc.load_gather(r_buf, [rows, jnp.full((L,), d, jnp.int32)])
                rf = plsc.load_gather(r_buf, [rows, jnp.full((L,), D + d, jnp.int32)])
                rs = plsc.load_gather(r_buf, [rows, jnp.full((L,), 2 * D + d, jnp.int32)])
                rb = plsc.load_gather(r_buf, [rows, jnp.full((L,), 3 * D + d, jnp.int32)])
                cc = cc * rf + rt
                off = jnp.abs(oo) * jnp.abs(rs) + jnp.abs(rb)
                delta = jnp.abs(cc - aa)
                dout = jnp.maximum(delta - off, jnp.float32(0.0))
                din = jnp.minimum(delta, off)
                acc_o = acc_o + dout * dout
                acc_i = acc_i + din * din
            dist = _sqrt16(acc_o) + jnp.float32(0.5) * _sqrt16(acc_i)
            out_v[pl.ds(j * CHUNK + g * L, L)] = jnp.float32(GAMMA) - dist
            return gcarry

        lax.fori_loop(0, GPC, group, 0)
        return carry

    lax.fori_loop(0, NCHUNK, chunk, 0)

    pltpu.sync_copy(out_v, out_hbm.at[pl.ds(base, BPW)])


@jax.jit
def _run(heads, rels, tails, cep, cop, rel_all):
    mesh = plsc.VectorSubcoreMesh(core_axis_name="c", subcore_axis_name="s")
    k = functools.partial(
        pl.kernel,
        out_type=jax.ShapeDtypeStruct((B,), jnp.float32),
        mesh=mesh,
        compiler_params=pltpu.CompilerParams(
            needs_layout_passes=False, use_tc_tiling_on_sc=True),
        scratch_types=[
            pltpu.VMEM((BPW,), jnp.int32),
            pltpu.VMEM((BPW,), jnp.int32),
            pltpu.VMEM((BPW,), jnp.int32),
            pltpu.VMEM((BPW,), jnp.int32),
            pltpu.VMEM((BPW,), jnp.int32),
            pltpu.VMEM((CHUNK, 4 * D), jnp.float32),
            pltpu.VMEM((CHUNK, 4 * D), jnp.float32),
            pltpu.VMEM((CHUNK, 4 * D), jnp.float32),
            pltpu.VMEM((CHUNK, 4 * D), jnp.float32),
            pltpu.VMEM((BPW,), jnp.float32),
            pltpu.SemaphoreType.DMA,
        ],
    )(_body)
    return k(heads, rels, tails, cep, cop, rel_all)


def kernel(heads, rels, tails, class_embed, class_offset, rel_embed,
           rel_factor, scale_embed, scale_bias):
    cep, cop = _repack(class_embed.T, class_offset.T)
    rel_all = jnp.concatenate(
        [rel_embed, rel_factor, scale_embed, scale_bias], axis=1)
    return _run(heads.astype(jnp.int32), rels.astype(jnp.int32),
                tails.astype(jnp.int32), cep, cop, rel_all)


# speedup vs baseline: 3.1520x; 2.3237x over previous
import functools

import jax
import jax.numpy as jnp
from jax import lax
from jax.experimental import pallas as pl
from jax.experimental.pallas import tpu as pltpu
from jax.experimental.pallas import tpu_sc as plsc

NB_CLASSES = 1000000
NB_RELS = 1000
D = 32
B = 16384
GAMMA = 10.0

NC, NS, L = 2, 16, 16
NW = NC * NS
BPW = B // NW
CHUNK = 128
NCHUNK = BPW // CHUNK
GPC = CHUNK // L

RB = 8192
RGRID = (NB_CLASSES + RB - 1) // RB
RROWS = RGRID * (RB // 4)
RB_LOG = RB.bit_length() - 1
Q_LOG = RB_LOG - 2
Q_MASK = (RB // 4) - 1


def _repack_body(ce_ref, co_ref, cep_ref, cop_ref):
    q = RB // 4
    xe = ce_ref[...]
    xo = co_ref[...]
    xs_e = jnp.concatenate([xe[:, a * q:(a + 1) * q] for a in range(4)], axis=0)
    xs_o = jnp.concatenate([xo[:, a * q:(a + 1) * q] for a in range(4)], axis=0)
    cep_ref[...] = jnp.transpose(xs_e)
    cop_ref[...] = jnp.transpose(xs_o)


def _repack(ceT, coT):
    return pl.pallas_call(
        _repack_body,
        grid=(RGRID,),
        in_specs=[pl.BlockSpec((D, RB), lambda i: (0, i)),
                  pl.BlockSpec((D, RB), lambda i: (0, i))],
        out_specs=[pl.BlockSpec((RB // 4, 4 * D), lambda i: (i, 0)),
                   pl.BlockSpec((RB // 4, 4 * D), lambda i: (i, 0))],
        out_shape=[jax.ShapeDtypeStruct((RROWS, 4 * D), jnp.float32),
                   jax.ShapeDtypeStruct((RROWS, 4 * D), jnp.float32)],
        compiler_params=pltpu.CompilerParams(
            fuse_transposed_lhs_in_matmul=True),
    )(ceT, coT)


def _sqrt16(x):
    xs = jnp.maximum(x, jnp.float32(1e-30))
    i = lax.bitcast_convert_type(xs, jnp.int32)
    i = jnp.int32(0x5F3759DF) - lax.shift_right_logical(i, 1)
    y = lax.bitcast_convert_type(i, jnp.float32)
    half = jnp.float32(0.5) * xs
    for _ in range(3):
        y = y * (jnp.float32(1.5) - half * y * y)
    return x * y


def _body(heads_hbm, rels_hbm, tails_hbm, cep_hbm, cop_hbm, rel_hbm, out_hbm,
          hv, rv, tv, hrow, trow, hc_buf, ho_buf, tc_buf, r_buf, out_v, sem):
    cid = lax.axis_index("c")
    sid = lax.axis_index("s")
    wid = sid * NC + cid
    base = wid * BPW

    pltpu.sync_copy(heads_hbm.at[pl.ds(base, BPW)], hv)
    pltpu.sync_copy(rels_hbm.at[pl.ds(base, BPW)], rv)
    pltpu.sync_copy(tails_hbm.at[pl.ds(base, BPW)], tv)

    def rowify(k, carry):
        sl = pl.ds(k * L, L)
        h = hv[sl]
        t = tv[sl]
        hrow[sl] = lax.shift_left(lax.shift_right_logical(h, RB_LOG), Q_LOG) + \
            jnp.bitwise_and(h, Q_MASK)
        trow[sl] = lax.shift_left(lax.shift_right_logical(t, RB_LOG), Q_LOG) + \
            jnp.bitwise_and(t, Q_MASK)
        return carry
    lax.fori_loop(0, BPW // L, rowify, 0)

    lanes = lax.iota(jnp.int32, L)

    def chunk(j, carry):
        isl = pl.ds(j * CHUNK, CHUNK)
        cp1 = pltpu.async_copy(cep_hbm.at[hrow.at[isl]], hc_buf, sem)
        cp2 = pltpu.async_copy(cop_hbm.at[hrow.at[isl]], ho_buf, sem)
        cp3 = pltpu.async_copy(cep_hbm.at[trow.at[isl]], tc_buf, sem)
        cp4 = pltpu.async_copy(rel_hbm.at[rv.at[isl]], r_buf, sem)
        cp1.wait()
        cp2.wait()
        cp3.wait()
        cp4.wait()

        def group(g, gcarry):
            rows = g * L + lanes
            sl = pl.ds(j * CHUNK + g * L, L)
            h = hv[sl]
            t = tv[sl]
            hq = lax.shift_left(
                jnp.bitwise_and(lax.shift_right_logical(h, Q_LOG), 3), 5)
            tq = lax.shift_left(
                jnp.bitwise_and(lax.shift_right_logical(t, Q_LOG), 3), 5)
            acc_o = jnp.zeros((L,), jnp.float32)
            acc_i = jnp.zeros((L,), jnp.float32)
            for d in range(D):
                cc = plsc.load_gather(hc_buf, [rows, hq + d])
                oo = plsc.load_gather(ho_buf, [rows, hq + d])
                aa = plsc.load_gather(tc_buf, [rows, tq + d])
                rt = plsc.load_gather(r_buf, [rows, jnp.full((L,), d, jnp.int32)])
                rf = plsc.load_gather(r_buf, [rows, jnp.full((L,), D + d, jnp.int32)])
                rs = plsc.load_gather(r_buf, [rows, jnp.full((L,), 2 * D + d, jnp.int32)])
                rb = plsc.load_gather(r_buf, [rows, jnp.full((L,), 3 * D + d, jnp.int32)])
                cc = cc * rf + rt
                off = jnp.abs(oo) * jnp.abs(rs) + jnp.abs(rb)
                delta = jnp.abs(cc - aa)
                dout = jnp.maximum(delta - off, jnp.float32(0.0))
                din = jnp.minimum(delta, off)
                acc_o = acc_o + dout * dout
                acc_i = acc_i + din * din
            dist = _sqrt16(acc_o) + jnp.float32(0.5) * _sqrt16(acc_i)
            out_v[pl.ds(j * CHUNK + g * L, L)] = jnp.float32(GAMMA) - dist
            return gcarry

        lax.fori_loop(0, GPC, group, 0)
        return carry

    lax.fori_loop(0, NCHUNK, chunk, 0)

    pltpu.sync_copy(out_v, out_hbm.at[pl.ds(base, BPW)])


@jax.jit
def _run(heads, rels, tails, cep, cop, rel_all):
    mesh = plsc.VectorSubcoreMesh(core_axis_name="c", subcore_axis_name="s")
    k = functools.partial(
        pl.kernel,
        out_type=jax.ShapeDtypeStruct((B,), jnp.float32),
        mesh=mesh,
        compiler_params=pltpu.CompilerParams(
            needs_layout_passes=False, use_tc_tiling_on_sc=True),
        scratch_types=[
            pltpu.VMEM((BPW,), jnp.int32),
            pltpu.VMEM((BPW,), jnp.int32),
            pltpu.VMEM((BPW,), jnp.int32),
            pltpu.VMEM((BPW,), jnp.int32),
            pltpu.VMEM((BPW,), jnp.int32),
            pltpu.VMEM((CHUNK, 4 * D), jnp.float32),
            pltpu.VMEM((CHUNK, 4 * D), jnp.float32),
            pltpu.VMEM((CHUNK, 4 * D), jnp.float32),
            pltpu.VMEM((CHUNK, 4 * D), jnp.float32),
            pltpu.VMEM((BPW,), jnp.float32),
            pltpu.SemaphoreType.DMA,
        ],
    )(_body)
    return k(heads, rels, tails, cep, cop, rel_all)


def kernel(heads, rels, tails, class_embed, class_offset, rel_embed,
           rel_factor, scale_embed, scale_bias):
    cep, cop = _repack(class_embed.T, class_offset.T)
    rel_all = jnp.concatenate(
        [rel_embed, rel_factor, scale_embed, scale_bias], axis=1)
    return _run(heads.astype(jnp.int32), rels.astype(jnp.int32),
                tails.astype(jnp.int32), cep, cop, rel_all)


# speedup vs baseline: 3.2799x; 1.0406x over previous
import functools

import jax
import jax.numpy as jnp
from jax import lax
from jax.experimental import pallas as pl
from jax.experimental.pallas import tpu as pltpu
from jax.experimental.pallas import tpu_sc as plsc

NB_CLASSES = 1000000
NB_RELS = 1000
D = 32
B = 16384
GAMMA = 10.0

NC, NS, L = 2, 16, 16
NW = NC * NS
BPW = B // NW
CHUNK = 64
NCHUNK = BPW // CHUNK
GPC = CHUNK // L

RB = 8192
RGRID = (NB_CLASSES + RB - 1) // RB
RROWS = RGRID * (RB // 4)
RB_LOG = RB.bit_length() - 1
Q_LOG = RB_LOG - 2
Q_MASK = (RB // 4) - 1


def _repack_body(ce_ref, co_ref, cep_ref, cop_ref):
    q = RB // 4
    xe = ce_ref[...]
    xo = co_ref[...]
    xs_e = jnp.concatenate([xe[:, a * q:(a + 1) * q] for a in range(4)], axis=0)
    xs_o = jnp.concatenate([xo[:, a * q:(a + 1) * q] for a in range(4)], axis=0)
    cep_ref[...] = jnp.transpose(xs_e)
    cop_ref[...] = jnp.transpose(xs_o)


def _repack(ceT, coT):
    return pl.pallas_call(
        _repack_body,
        grid=(RGRID,),
        in_specs=[pl.BlockSpec((D, RB), lambda i: (0, i)),
                  pl.BlockSpec((D, RB), lambda i: (0, i))],
        out_specs=[pl.BlockSpec((RB // 4, 4 * D), lambda i: (i, 0)),
                   pl.BlockSpec((RB // 4, 4 * D), lambda i: (i, 0))],
        out_shape=[jax.ShapeDtypeStruct((RROWS, 4 * D), jnp.float32),
                   jax.ShapeDtypeStruct((RROWS, 4 * D), jnp.float32)],
        compiler_params=pltpu.CompilerParams(
            fuse_transposed_lhs_in_matmul=True),
    )(ceT, coT)


def _sqrt16(x):
    xs = jnp.maximum(x, jnp.float32(1e-30))
    i = lax.bitcast_convert_type(xs, jnp.int32)
    i = jnp.int32(0x5F3759DF) - lax.shift_right_logical(i, 1)
    y = lax.bitcast_convert_type(i, jnp.float32)
    half = jnp.float32(0.5) * xs
    for _ in range(3):
        y = y * (jnp.float32(1.5) - half * y * y)
    return x * y


def _body(heads_hbm, rels_hbm, tails_hbm, cep_hbm, cop_hbm, rel_hbm, out_hbm,
          hv, rv, tv, hrow, trow, hc0, ho0, tc0, r0, hc1, ho1, tc1, r1,
          out_v, sem0, sem1):
    cid = lax.axis_index("c")
    sid = lax.axis_index("s")
    wid = sid * NC + cid
    base = wid * BPW

    pltpu.sync_copy(heads_hbm.at[pl.ds(base, BPW)], hv)
    pltpu.sync_copy(rels_hbm.at[pl.ds(base, BPW)], rv)
    pltpu.sync_copy(tails_hbm.at[pl.ds(base, BPW)], tv)

    def rowify(k, carry):
        sl = pl.ds(k * L, L)
        h = hv[sl]
        t = tv[sl]
        hrow[sl] = lax.shift_left(lax.shift_right_logical(h, RB_LOG), Q_LOG) + \
            jnp.bitwise_and(h, Q_MASK)
        trow[sl] = lax.shift_left(lax.shift_right_logical(t, RB_LOG), Q_LOG) + \
            jnp.bitwise_and(t, Q_MASK)
        return carry
    lax.fori_loop(0, BPW // L, rowify, 0)

    lanes = lax.iota(jnp.int32, L)
    bufsets = ((hc0, ho0, tc0, r0, sem0), (hc1, ho1, tc1, r1, sem1))

    def fire(c, bs):
        hc_buf, ho_buf, tc_buf, r_buf, sem = bs
        isl = pl.ds(c * CHUNK, CHUNK)
        pltpu.async_copy(cep_hbm.at[hrow.at[isl]], hc_buf, sem)
        pltpu.async_copy(cop_hbm.at[hrow.at[isl]], ho_buf, sem)
        pltpu.async_copy(cep_hbm.at[trow.at[isl]], tc_buf, sem)
        pltpu.async_copy(rel_hbm.at[rv.at[isl]], r_buf, sem)

    def drain(c, bs):
        hc_buf, ho_buf, tc_buf, r_buf, sem = bs
        isl = pl.ds(c * CHUNK, CHUNK)
        pltpu.make_async_copy(cep_hbm.at[hrow.at[isl]], hc_buf, sem).wait()
        pltpu.make_async_copy(cop_hbm.at[hrow.at[isl]], ho_buf, sem).wait()
        pltpu.make_async_copy(cep_hbm.at[trow.at[isl]], tc_buf, sem).wait()
        pltpu.make_async_copy(rel_hbm.at[rv.at[isl]], r_buf, sem).wait()

    def compute(j, bs):
        hc_buf, ho_buf, tc_buf, r_buf, _ = bs

        def group(g, gcarry):
            rows = g * L + lanes
            sl = pl.ds(j * CHUNK + g * L, L)
            h = hv[sl]
            t = tv[sl]
            hq = lax.shift_left(
                jnp.bitwise_and(lax.shift_right_logical(h, Q_LOG), 3), 5)
            tq = lax.shift_left(
                jnp.bitwise_and(lax.shift_right_logical(t, Q_LOG), 3), 5)
            acc_o = jnp.zeros((L,), jnp.float32)
            acc_i = jnp.zeros((L,), jnp.float32)
            for d in range(D):
                cc = plsc.load_gather(hc_buf, [rows, hq + d])
                oo = plsc.load_gather(ho_buf, [rows, hq + d])
                aa = plsc.load_gather(tc_buf, [rows, tq + d])
                rt = plsc.load_gather(r_buf, [rows, jnp.full((L,), d, jnp.int32)])
                rf = plsc.load_gather(r_buf, [rows, jnp.full((L,), D + d, jnp.int32)])
                rs = plsc.load_gather(r_buf, [rows, jnp.full((L,), 2 * D + d, jnp.int32)])
                rb = plsc.load_gather(r_buf, [rows, jnp.full((L,), 3 * D + d, jnp.int32)])
                cc = cc * rf + rt
                off = jnp.abs(oo) * jnp.abs(rs) + jnp.abs(rb)
                delta = jnp.abs(cc - aa)
                dout = jnp.maximum(delta - off, jnp.float32(0.0))
                din = jnp.minimum(delta, off)
                acc_o = acc_o + dout * dout
                acc_i = acc_i + din * din
            dist = _sqrt16(acc_o) + jnp.float32(0.5) * _sqrt16(acc_i)
            out_v[pl.ds(j * CHUNK + g * L, L)] = jnp.float32(GAMMA) - dist
            return gcarry

        lax.fori_loop(0, GPC, group, 0)

    fire(0, bufsets[0])

    def pair(j2, carry):
        c0 = j2 * 2
        fire(c0 + 1, bufsets[1])
        drain(c0, bufsets[0])
        compute(c0, bufsets[0])

        @pl.when(j2 < NCHUNK // 2 - 1)
        def _():
            fire(c0 + 2, bufsets[0])

        drain(c0 + 1, bufsets[1])
        compute(c0 + 1, bufsets[1])
        return carry

    lax.fori_loop(0, NCHUNK // 2, pair, 0)

    pltpu.sync_copy(out_v, out_hbm.at[pl.ds(base, BPW)])


@jax.jit
def _run(heads, rels, tails, cep, cop, rel_all):
    mesh = plsc.VectorSubcoreMesh(core_axis_name="c", subcore_axis_name="s")
    k = functools.partial(
        pl.kernel,
        out_type=jax.ShapeDtypeStruct((B,), jnp.float32),
        mesh=mesh,
        compiler_params=pltpu.CompilerParams(
            needs_layout_passes=False, use_tc_tiling_on_sc=True),
        scratch_types=[
            pltpu.VMEM((BPW,), jnp.int32),
            pltpu.VMEM((BPW,), jnp.int32),
            pltpu.VMEM((BPW,), jnp.int32),
            pltpu.VMEM((BPW,), jnp.int32),
            pltpu.VMEM((BPW,), jnp.int32),
            pltpu.VMEM((CHUNK, 4 * D), jnp.float32),
            pltpu.VMEM((CHUNK, 4 * D), jnp.float32),
            pltpu.VMEM((CHUNK, 4 * D), jnp.float32),
            pltpu.VMEM((CHUNK, 4 * D), jnp.float32),
            pltpu.VMEM((CHUNK, 4 * D), jnp.float32),
            pltpu.VMEM((CHUNK, 4 * D), jnp.float32),
            pltpu.VMEM((CHUNK, 4 * D), jnp.float32),
            pltpu.VMEM((CHUNK, 4 * D), jnp.float32),
            pltpu.VMEM((BPW,), jnp.float32),
            pltpu.SemaphoreType.DMA,
            pltpu.SemaphoreType.DMA,
        ],
    )(_body)
    return k(heads, rels, tails, cep, cop, rel_all)


def kernel(heads, rels, tails, class_embed, class_offset, rel_embed,
           rel_factor, scale_embed, scale_bias):
    cep, cop = _repack(class_embed.T, class_offset.T)
    rel_all = jnp.concatenate(
        [rel_embed, rel_factor, scale_embed, scale_bias], axis=1)
    return _run(heads.astype(jnp.int32), rels.astype(jnp.int32),
                tails.astype(jnp.int32), cep, cop, rel_all)


# speedup vs baseline: 3.6206x; 1.1039x over previous
import functools

import jax
import jax.numpy as jnp
from jax import lax
from jax.experimental import pallas as pl
from jax.experimental.pallas import tpu as pltpu
from jax.experimental.pallas import tpu_sc as plsc

NB_CLASSES = 1000000
NB_RELS = 1000
D = 32
B = 16384
GAMMA = 10.0

NC, NS, L = 2, 16, 16
NW = NC * NS
BPW = B // NW
CHUNK = 64
NCHUNK = BPW // CHUNK
GPC = CHUNK // L

RB = 8192
RGRID = (NB_CLASSES + RB - 1) // RB
RB8 = RB // 8
RROWS = RGRID * RB8
RB_LOG = RB.bit_length() - 1
E_LOG = RB_LOG - 3
E_MASK = RB8 - 1


def _repack_body(ce_ref, co_ref, cep_ref, cop_ref):
    def pack(ref):
        x = ref[...]
        xs = jnp.concatenate(
            [x[:, a * RB8:(a + 1) * RB8] for a in range(8)], axis=0)
        t = jnp.transpose(xs)
        v = lax.bitcast_convert_type(t, jnp.int32)
        r = v + jnp.int32(0x7FFF) + \
            jnp.bitwise_and(lax.shift_right_logical(v, 16), 1)
        hi = jnp.bitwise_and(r[:, :4 * D], jnp.int32(-65536))
        lo = lax.shift_right_logical(r[:, 4 * D:], 16)
        return jnp.bitwise_or(hi, lo)

    cep_ref[...] = pack(ce_ref)
    cop_ref[...] = pack(co_ref)


def _repack(ceT, coT):
    return pl.pallas_call(
        _repack_body,
        grid=(RGRID,),
        in_specs=[pl.BlockSpec((D, RB), lambda i: (0, i)),
                  pl.BlockSpec((D, RB), lambda i: (0, i))],
        out_specs=[pl.BlockSpec((RB8, 4 * D), lambda i: (i, 0)),
                   pl.BlockSpec((RB8, 4 * D), lambda i: (i, 0))],
        out_shape=[jax.ShapeDtypeStruct((RROWS, 4 * D), jnp.int32),
                   jax.ShapeDtypeStruct((RROWS, 4 * D), jnp.int32)],
        compiler_params=pltpu.CompilerParams(
            fuse_transposed_lhs_in_matmul=True),
    )(ceT, coT)


def _sqrt16(x):
    xs = jnp.maximum(x, jnp.float32(1e-30))
    i = lax.bitcast_convert_type(xs, jnp.int32)
    i = jnp.int32(0x5F3759DF) - lax.shift_right_logical(i, 1)
    y = lax.bitcast_convert_type(i, jnp.float32)
    half = jnp.float32(0.5) * xs
    for _ in range(3):
        y = y * (jnp.float32(1.5) - half * y * y)
    return x * y


def _body(heads_hbm, rels_hbm, tails_hbm, cep_hbm, cop_hbm, rel_hbm, out_hbm,
          hv, rv, tv, hrow, trow, hc0, ho0, tc0, r0, hc1, ho1, tc1, r1,
          out_v, sem0, sem1):
    cid = lax.axis_index("c")
    sid = lax.axis_index("s")
    wid = sid * NC + cid
    base = wid * BPW

    pltpu.sync_copy(heads_hbm.at[pl.ds(base, BPW)], hv)
    pltpu.sync_copy(rels_hbm.at[pl.ds(base, BPW)], rv)
    pltpu.sync_copy(tails_hbm.at[pl.ds(base, BPW)], tv)

    def rowify(k, carry):
        sl = pl.ds(k * L, L)
        h = hv[sl]
        t = tv[sl]
        hrow[sl] = lax.shift_left(lax.shift_right_logical(h, RB_LOG), E_LOG) + \
            jnp.bitwise_and(h, E_MASK)
        trow[sl] = lax.shift_left(lax.shift_right_logical(t, RB_LOG), E_LOG) + \
            jnp.bitwise_and(t, E_MASK)
        return carry
    lax.fori_loop(0, BPW // L, rowify, 0)

    lanes = lax.iota(jnp.int32, L)
    bufsets = ((hc0, ho0, tc0, r0, sem0), (hc1, ho1, tc1, r1, sem1))

    def fire(c, bs):
        hc_buf, ho_buf, tc_buf, r_buf, sem = bs
        isl = pl.ds(c * CHUNK, CHUNK)
        pltpu.async_copy(cep_hbm.at[hrow.at[isl]], hc_buf, sem)
        pltpu.async_copy(cop_hbm.at[hrow.at[isl]], ho_buf, sem)
        pltpu.async_copy(cep_hbm.at[trow.at[isl]], tc_buf, sem)
        pltpu.async_copy(rel_hbm.at[rv.at[isl]], r_buf, sem)

    def drain(c, bs):
        hc_buf, ho_buf, tc_buf, r_buf, sem = bs
        isl = pl.ds(c * CHUNK, CHUNK)
        pltpu.make_async_copy(cep_hbm.at[hrow.at[isl]], hc_buf, sem).wait()
        pltpu.make_async_copy(cop_hbm.at[hrow.at[isl]], ho_buf, sem).wait()
        pltpu.make_async_copy(cep_hbm.at[trow.at[isl]], tc_buf, sem).wait()
        pltpu.make_async_copy(rel_hbm.at[rv.at[isl]], r_buf, sem).wait()

    def compute(j, bs):
        hc_buf, ho_buf, tc_buf, r_buf, _ = bs

        def group(g, gcarry):
            rows = g * L + lanes
            sl = pl.ds(j * CHUNK + g * L, L)
            h = hv[sl]
            t = tv[sl]
            hs = jnp.bitwise_and(lax.shift_right_logical(h, E_LOG), 7)
            ts = jnp.bitwise_and(lax.shift_right_logical(t, E_LOG), 7)
            hq = lax.shift_left(jnp.bitwise_and(hs, 3), 5)
            tq = lax.shift_left(jnp.bitwise_and(ts, 3), 5)
            h_hi = hs < 4
            t_hi = ts < 4

            def widen(xi, hi_mask):
                bits = jnp.where(hi_mask, jnp.bitwise_and(xi, jnp.int32(-65536)),
                                 lax.shift_left(xi, 16))
                return lax.bitcast_convert_type(bits, jnp.float32)

            acc_o = jnp.zeros((L,), jnp.float32)
            acc_i = jnp.zeros((L,), jnp.float32)
            for d in range(D):
                cc = widen(plsc.load_gather(hc_buf, [rows, hq + d]), h_hi)
                oo = widen(plsc.load_gather(ho_buf, [rows, hq + d]), h_hi)
                aa = widen(plsc.load_gather(tc_buf, [rows, tq + d]), t_hi)
                rt = plsc.load_gather(r_buf, [rows, jnp.full((L,), d, jnp.int32)])
                rf = plsc.load_gather(r_buf, [rows, jnp.full((L,), D + d, jnp.int32)])
                rs = plsc.load_gather(r_buf, [rows, jnp.full((L,), 2 * D + d, jnp.int32)])
                rb = plsc.load_gather(r_buf, [rows, jnp.full((L,), 3 * D + d, jnp.int32)])
                cc = cc * rf + rt
                off = jnp.abs(oo) * jnp.abs(rs) + jnp.abs(rb)
                delta = jnp.abs(cc - aa)
                dout = jnp.maximum(delta - off, jnp.float32(0.0))
                din = jnp.minimum(delta, off)
                acc_o = acc_o + dout * dout
                acc_i = acc_i + din * din
            dist = _sqrt16(acc_o) + jnp.float32(0.5) * _sqrt16(acc_i)
            out_v[pl.ds(j * CHUNK + g * L, L)] = jnp.float32(GAMMA) - dist
            return gcarry

        lax.fori_loop(0, GPC, group, 0)

    fire(0, bufsets[0])

    def pair(j2, carry):
        c0 = j2 * 2
        fire(c0 + 1, bufsets[1])
        drain(c0, bufsets[0])
        compute(c0, bufsets[0])

        @pl.when(j2 < NCHUNK // 2 - 1)
        def _():
            fire(c0 + 2, bufsets[0])

        drain(c0 + 1, bufsets[1])
        compute(c0 + 1, bufsets[1])
        return carry

    lax.fori_loop(0, NCHUNK // 2, pair, 0)

    pltpu.sync_copy(out_v, out_hbm.at[pl.ds(base, BPW)])


@jax.jit
def _run(heads, rels, tails, cep, cop, rel_all):
    mesh = plsc.VectorSubcoreMesh(core_axis_name="c", subcore_axis_name="s")
    k = functools.partial(
        pl.kernel,
        out_type=jax.ShapeDtypeStruct((B,), jnp.float32),
        mesh=mesh,
        compiler_params=pltpu.CompilerParams(
            needs_layout_passes=False, use_tc_tiling_on_sc=True),
        scratch_types=[
            pltpu.VMEM((BPW,), jnp.int32),
            pltpu.VMEM((BPW,), jnp.int32),
            pltpu.VMEM((BPW,), jnp.int32),
            pltpu.VMEM((BPW,), jnp.int32),
            pltpu.VMEM((BPW,), jnp.int32),
            pltpu.VMEM((CHUNK, 4 * D), jnp.int32),
            pltpu.VMEM((CHUNK, 4 * D), jnp.int32),
            pltpu.VMEM((CHUNK, 4 * D), jnp.int32),
            pltpu.VMEM((CHUNK, 4 * D), jnp.float32),
            pltpu.VMEM((CHUNK, 4 * D), jnp.int32),
            pltpu.VMEM((CHUNK, 4 * D), jnp.int32),
            pltpu.VMEM((CHUNK, 4 * D), jnp.int32),
            pltpu.VMEM((CHUNK, 4 * D), jnp.float32),
            pltpu.VMEM((BPW,), jnp.float32),
            pltpu.SemaphoreType.DMA,
            pltpu.SemaphoreType.DMA,
        ],
    )(_body)
    return k(heads, rels, tails, cep, cop, rel_all)


def kernel(heads, rels, tails, class_embed, class_offset, rel_embed,
           rel_factor, scale_embed, scale_bias):
    cep, cop = _repack(class_embed.T, class_offset.T)
    rel_all = jnp.concatenate(
        [rel_embed, rel_factor, scale_embed, scale_bias], axis=1)
    return _run(heads.astype(jnp.int32), rels.astype(jnp.int32),
                tails.astype(jnp.int32), cep, cop, rel_all)


# speedup vs baseline: 4.1895x; 1.1571x over previous
import functools

import jax
import jax.numpy as jnp
from jax import lax
from jax.experimental import pallas as pl
from jax.experimental.pallas import tpu as pltpu
from jax.experimental.pallas import tpu_sc as plsc

NB_CLASSES = 1000000
NB_RELS = 1000
D = 32
B = 16384
GAMMA = 10.0

NC, NS, L = 2, 16, 16
NW = NC * NS
BPW = B // NW
CHUNK = 64
NCHUNK = BPW // CHUNK
GPC = CHUNK // L

RB = 16384
RGRID = (NB_CLASSES + RB - 1) // RB
RB8 = RB // 8
RROWS = RGRID * RB8
RB_LOG = RB.bit_length() - 1
E_LOG = RB_LOG - 3
E_MASK = RB8 - 1


def _repack_body(ce_ref, co_ref, cep_ref, cop_ref):
    def pack(ref):
        x = ref[...]
        xs = jnp.concatenate(
            [x[:, a * RB8:(a + 1) * RB8] for a in range(8)], axis=0)
        t = jnp.transpose(xs)
        v = lax.bitcast_convert_type(t, jnp.int32)
        r = v + jnp.int32(0x7FFF) + \
            jnp.bitwise_and(lax.shift_right_logical(v, 16), 1)
        hi = jnp.bitwise_and(r[:, :4 * D], jnp.int32(-65536))
        lo = lax.shift_right_logical(r[:, 4 * D:], 16)
        return jnp.bitwise_or(hi, lo)

    cep_ref[...] = pack(ce_ref)
    cop_ref[...] = pack(co_ref)


def _repack(ceT, coT):
    return pl.pallas_call(
        _repack_body,
        grid=(RGRID,),
        in_specs=[pl.BlockSpec((D, RB), lambda i: (0, i)),
                  pl.BlockSpec((D, RB), lambda i: (0, i))],
        out_specs=[pl.BlockSpec((RB8, 4 * D), lambda i: (i, 0)),
                   pl.BlockSpec((RB8, 4 * D), lambda i: (i, 0))],
        out_shape=[jax.ShapeDtypeStruct((RROWS, 4 * D), jnp.int32),
                   jax.ShapeDtypeStruct((RROWS, 4 * D), jnp.int32)],
        compiler_params=pltpu.CompilerParams(
            fuse_transposed_lhs_in_matmul=True),
    )(ceT, coT)


def _sqrt16(x):
    xs = jnp.maximum(x, jnp.float32(1e-30))
    i = lax.bitcast_convert_type(xs, jnp.int32)
    i = jnp.int32(0x5F3759DF) - lax.shift_right_logical(i, 1)
    y = lax.bitcast_convert_type(i, jnp.float32)
    half = jnp.float32(0.5) * xs
    for _ in range(3):
        y = y * (jnp.float32(1.5) - half * y * y)
    return x * y


def _body(heads_hbm, rels_hbm, tails_hbm, cep_hbm, cop_hbm, rel_hbm, out_hbm,
          hv, rv, tv, hrow, trow, hc0, ho0, tc0, r0, hc1, ho1, tc1, r1,
          out_v, sem0, sem1):
    cid = lax.axis_index("c")
    sid = lax.axis_index("s")
    wid = sid * NC + cid
    base = wid * BPW

    pltpu.sync_copy(heads_hbm.at[pl.ds(base, BPW)], hv)
    pltpu.sync_copy(rels_hbm.at[pl.ds(base, BPW)], rv)
    pltpu.sync_copy(tails_hbm.at[pl.ds(base, BPW)], tv)

    def rowify(k, carry):
        sl = pl.ds(k * L, L)
        h = hv[sl]
        t = tv[sl]
        hrow[sl] = lax.shift_left(lax.shift_right_logical(h, RB_LOG), E_LOG) + \
            jnp.bitwise_and(h, E_MASK)
        trow[sl] = lax.shift_left(lax.shift_right_logical(t, RB_LOG), E_LOG) + \
            jnp.bitwise_and(t, E_MASK)
        return carry
    lax.fori_loop(0, BPW // L, rowify, 0)

    lanes = lax.iota(jnp.int32, L)
    bufsets = ((hc0, ho0, tc0, r0, sem0), (hc1, ho1, tc1, r1, sem1))

    def fire(c, bs):
        hc_buf, ho_buf, tc_buf, r_buf, sem = bs
        isl = pl.ds(c * CHUNK, CHUNK)
        pltpu.async_copy(cep_hbm.at[hrow.at[isl]], hc_buf, sem)
        pltpu.async_copy(cop_hbm.at[hrow.at[isl]], ho_buf, sem)
        pltpu.async_copy(cep_hbm.at[trow.at[isl]], tc_buf, sem)
        pltpu.async_copy(rel_hbm.at[rv.at[isl]], r_buf, sem)

    def drain(c, bs):
        hc_buf, ho_buf, tc_buf, r_buf, sem = bs
        isl = pl.ds(c * CHUNK, CHUNK)
        pltpu.make_async_copy(cep_hbm.at[hrow.at[isl]], hc_buf, sem).wait()
        pltpu.make_async_copy(cop_hbm.at[hrow.at[isl]], ho_buf, sem).wait()
        pltpu.make_async_copy(cep_hbm.at[trow.at[isl]], tc_buf, sem).wait()
        pltpu.make_async_copy(rel_hbm.at[rv.at[isl]], r_buf, sem).wait()

    def compute(j, bs):
        hc_buf, ho_buf, tc_buf, r_buf, _ = bs

        def group(g, gcarry):
            rows = g * L + lanes
            sl = pl.ds(j * CHUNK + g * L, L)
            h = hv[sl]
            t = tv[sl]
            hs = jnp.bitwise_and(lax.shift_right_logical(h, E_LOG), 7)
            ts = jnp.bitwise_and(lax.shift_right_logical(t, E_LOG), 7)
            hq = lax.shift_left(jnp.bitwise_and(hs, 3), 5)
            tq = lax.shift_left(jnp.bitwise_and(ts, 3), 5)
            h_hi = hs < 4
            t_hi = ts < 4

            def widen(xi, hi_mask):
                bits = jnp.where(hi_mask, jnp.bitwise_and(xi, jnp.int32(-65536)),
                                 lax.shift_left(xi, 16))
                return lax.bitcast_convert_type(bits, jnp.float32)

            acc_o = jnp.zeros((L,), jnp.float32)
            acc_i = jnp.zeros((L,), jnp.float32)
            for d in range(D):
                cc = widen(plsc.load_gather(hc_buf, [rows, hq + d]), h_hi)
                oo = widen(plsc.load_gather(ho_buf, [rows, hq + d]), h_hi)
                aa = widen(plsc.load_gather(tc_buf, [rows, tq + d]), t_hi)
                rt = plsc.load_gather(r_buf, [rows, jnp.full((L,), d, jnp.int32)])
                rf = plsc.load_gather(r_buf, [rows, jnp.full((L,), D + d, jnp.int32)])
                rs = plsc.load_gather(r_buf, [rows, jnp.full((L,), 2 * D + d, jnp.int32)])
                rb = plsc.load_gather(r_buf, [rows, jnp.full((L,), 3 * D + d, jnp.int32)])
                cc = cc * rf + rt
                off = jnp.abs(oo) * jnp.abs(rs) + jnp.abs(rb)
                delta = jnp.abs(cc - aa)
                dout = jnp.maximum(delta - off, jnp.float32(0.0))
                din = jnp.minimum(delta, off)
                acc_o = acc_o + dout * dout
                acc_i = acc_i + din * din
            dist = _sqrt16(acc_o) + jnp.float32(0.5) * _sqrt16(acc_i)
            out_v[pl.ds(j * CHUNK + g * L, L)] = jnp.float32(GAMMA) - dist
            return gcarry

        lax.fori_loop(0, GPC, group, 0)

    fire(0, bufsets[0])

    def pair(j2, carry):
        c0 = j2 * 2
        fire(c0 + 1, bufsets[1])
        drain(c0, bufsets[0])
        compute(c0, bufsets[0])

        @pl.when(j2 < NCHUNK // 2 - 1)
        def _():
            fire(c0 + 2, bufsets[0])

        drain(c0 + 1, bufsets[1])
        compute(c0 + 1, bufsets[1])
        return carry

    lax.fori_loop(0, NCHUNK // 2, pair, 0)

    pltpu.sync_copy(out_v, out_hbm.at[pl.ds(base, BPW)])


@jax.jit
def _run(heads, rels, tails, cep, cop, rel_all):
    mesh = plsc.VectorSubcoreMesh(core_axis_name="c", subcore_axis_name="s")
    k = functools.partial(
        pl.kernel,
        out_type=jax.ShapeDtypeStruct((B,), jnp.float32),
        mesh=mesh,
        compiler_params=pltpu.CompilerParams(
            needs_layout_passes=False, use_tc_tiling_on_sc=True),
        scratch_types=[
            pltpu.VMEM((BPW,), jnp.int32),
            pltpu.VMEM((BPW,), jnp.int32),
            pltpu.VMEM((BPW,), jnp.int32),
            pltpu.VMEM((BPW,), jnp.int32),
            pltpu.VMEM((BPW,), jnp.int32),
            pltpu.VMEM((CHUNK, 4 * D), jnp.int32),
            pltpu.VMEM((CHUNK, 4 * D), jnp.int32),
            pltpu.VMEM((CHUNK, 4 * D), jnp.int32),
            pltpu.VMEM((CHUNK, 4 * D), jnp.float32),
            pltpu.VMEM((CHUNK, 4 * D), jnp.int32),
            pltpu.VMEM((CHUNK, 4 * D), jnp.int32),
            pltpu.VMEM((CHUNK, 4 * D), jnp.int32),
            pltpu.VMEM((CHUNK, 4 * D), jnp.float32),
            pltpu.VMEM((BPW,), jnp.float32),
            pltpu.SemaphoreType.DMA,
            pltpu.SemaphoreType.DMA,
        ],
    )(_body)
    return k(heads, rels, tails, cep, cop, rel_all)


def kernel(heads, rels, tails, class_embed, class_offset, rel_embed,
           rel_factor, scale_embed, scale_bias):
    cep, cop = _repack(class_embed.T, class_offset.T)
    rel_all = jnp.concatenate(
        [rel_embed, rel_factor, scale_embed, scale_bias], axis=1)
    return _run(heads.astype(jnp.int32), rels.astype(jnp.int32),
                tails.astype(jnp.int32), cep, cop, rel_all)


# speedup vs baseline: 4.4534x; 1.0630x over previous
import functools

import jax
import jax.numpy as jnp
from jax import lax
from jax.experimental import pallas as pl
from jax.experimental.pallas import tpu as pltpu
from jax.experimental.pallas import tpu_sc as plsc

NB_CLASSES = 1000000
NB_RELS = 1000
D = 32
B = 16384
GAMMA = 10.0

NC, NS, L = 2, 16, 16
NW = NC * NS
BPW = B // NW
CHUNK = 64
NCHUNK = BPW // CHUNK
GPC = CHUNK // L

RB = 32768
RGRID = (NB_CLASSES + RB - 1) // RB
RB8 = RB // 8
RROWS = RGRID * RB8
RB_LOG = RB.bit_length() - 1
E_LOG = RB_LOG - 3
E_MASK = RB8 - 1


def _repack_body(ce_ref, co_ref, cep_ref, cop_ref):
    def pack(ref):
        x = ref[...]
        xs = jnp.concatenate(
            [x[:, a * RB8:(a + 1) * RB8] for a in range(8)], axis=0)
        t = jnp.transpose(xs)
        v = lax.bitcast_convert_type(t, jnp.int32)
        r = v + jnp.int32(0x7FFF) + \
            jnp.bitwise_and(lax.shift_right_logical(v, 16), 1)
        hi = jnp.bitwise_and(r[:, :4 * D], jnp.int32(-65536))
        lo = lax.shift_right_logical(r[:, 4 * D:], 16)
        return jnp.bitwise_or(hi, lo)

    cep_ref[...] = pack(ce_ref)
    cop_ref[...] = pack(co_ref)


def _repack(ceT, coT):
    return pl.pallas_call(
        _repack_body,
        grid=(RGRID,),
        in_specs=[pl.BlockSpec((D, RB), lambda i: (0, i)),
                  pl.BlockSpec((D, RB), lambda i: (0, i))],
        out_specs=[pl.BlockSpec((RB8, 4 * D), lambda i: (i, 0)),
                   pl.BlockSpec((RB8, 4 * D), lambda i: (i, 0))],
        out_shape=[jax.ShapeDtypeStruct((RROWS, 4 * D), jnp.int32),
                   jax.ShapeDtypeStruct((RROWS, 4 * D), jnp.int32)],
        compiler_params=pltpu.CompilerParams(
            fuse_transposed_lhs_in_matmul=True),
    )(ceT, coT)


def _sqrt16(x):
    xs = jnp.maximum(x, jnp.float32(1e-30))
    i = lax.bitcast_convert_type(xs, jnp.int32)
    i = jnp.int32(0x5F3759DF) - lax.shift_right_logical(i, 1)
    y = lax.bitcast_convert_type(i, jnp.float32)
    half = jnp.float32(0.5) * xs
    for _ in range(3):
        y = y * (jnp.float32(1.5) - half * y * y)
    return x * y


def _body(heads_hbm, rels_hbm, tails_hbm, cep_hbm, cop_hbm, rel_hbm, out_hbm,
          hv, rv, tv, hrow, trow, hc0, ho0, tc0, r0, hc1, ho1, tc1, r1,
          out_v, sem0, sem1):
    cid = lax.axis_index("c")
    sid = lax.axis_index("s")
    wid = sid * NC + cid
    base = wid * BPW

    pltpu.sync_copy(heads_hbm.at[pl.ds(base, BPW)], hv)
    pltpu.sync_copy(rels_hbm.at[pl.ds(base, BPW)], rv)
    pltpu.sync_copy(tails_hbm.at[pl.ds(base, BPW)], tv)

    def rowify(k, carry):
        sl = pl.ds(k * L, L)
        h = hv[sl]
        t = tv[sl]
        hrow[sl] = lax.shift_left(lax.shift_right_logical(h, RB_LOG), E_LOG) + \
            jnp.bitwise_and(h, E_MASK)
        trow[sl] = lax.shift_left(lax.shift_right_logical(t, RB_LOG), E_LOG) + \
            jnp.bitwise_and(t, E_MASK)
        return carry
    lax.fori_loop(0, BPW // L, rowify, 0)

    lanes = lax.iota(jnp.int32, L)
    bufsets = ((hc0, ho0, tc0, r0, sem0), (hc1, ho1, tc1, r1, sem1))

    def fire(c, bs):
        hc_buf, ho_buf, tc_buf, r_buf, sem = bs
        isl = pl.ds(c * CHUNK, CHUNK)
        pltpu.async_copy(cep_hbm.at[hrow.at[isl]], hc_buf, sem)
        pltpu.async_copy(cop_hbm.at[hrow.at[isl]], ho_buf, sem)
        pltpu.async_copy(cep_hbm.at[trow.at[isl]], tc_buf, sem)
        pltpu.async_copy(rel_hbm.at[rv.at[isl]], r_buf, sem)

    def drain(c, bs):
        hc_buf, ho_buf, tc_buf, r_buf, sem = bs
        isl = pl.ds(c * CHUNK, CHUNK)
        pltpu.make_async_copy(cep_hbm.at[hrow.at[isl]], hc_buf, sem).wait()
        pltpu.make_async_copy(cop_hbm.at[hrow.at[isl]], ho_buf, sem).wait()
        pltpu.make_async_copy(cep_hbm.at[trow.at[isl]], tc_buf, sem).wait()
        pltpu.make_async_copy(rel_hbm.at[rv.at[isl]], r_buf, sem).wait()

    def compute(j, bs):
        hc_buf, ho_buf, tc_buf, r_buf, _ = bs

        def group(g, gcarry):
            rows = g * L + lanes
            sl = pl.ds(j * CHUNK + g * L, L)
            h = hv[sl]
            t = tv[sl]
            hs = jnp.bitwise_and(lax.shift_right_logical(h, E_LOG), 7)
            ts = jnp.bitwise_and(lax.shift_right_logical(t, E_LOG), 7)
            hq = lax.shift_left(jnp.bitwise_and(hs, 3), 5)
            tq = lax.shift_left(jnp.bitwise_and(ts, 3), 5)
            h_hi = hs < 4
            t_hi = ts < 4

            def widen(xi, hi_mask):
                bits = jnp.where(hi_mask, jnp.bitwise_and(xi, jnp.int32(-65536)),
                                 lax.shift_left(xi, 16))
                return lax.bitcast_convert_type(bits, jnp.float32)

            acc_o = jnp.zeros((L,), jnp.float32)
            acc_i = jnp.zeros((L,), jnp.float32)
            for d in range(D):
                cc = widen(plsc.load_gather(hc_buf, [rows, hq + d]), h_hi)
                oo = widen(plsc.load_gather(ho_buf, [rows, hq + d]), h_hi)
                aa = widen(plsc.load_gather(tc_buf, [rows, tq + d]), t_hi)
                rt = plsc.load_gather(r_buf, [rows, jnp.full((L,), d, jnp.int32)])
                rf = plsc.load_gather(r_buf, [rows, jnp.full((L,), D + d, jnp.int32)])
                rs = plsc.load_gather(r_buf, [rows, jnp.full((L,), 2 * D + d, jnp.int32)])
                rb = plsc.load_gather(r_buf, [rows, jnp.full((L,), 3 * D + d, jnp.int32)])
                cc = cc * rf + rt
                off = jnp.abs(oo) * jnp.abs(rs) + jnp.abs(rb)
                delta = jnp.abs(cc - aa)
                dout = jnp.maximum(delta - off, jnp.float32(0.0))
                din = jnp.minimum(delta, off)
                acc_o = acc_o + dout * dout
                acc_i = acc_i + din * din
            dist = _sqrt16(acc_o) + jnp.float32(0.5) * _sqrt16(acc_i)
            out_v[pl.ds(j * CHUNK + g * L, L)] = jnp.float32(GAMMA) - dist
            return gcarry

        lax.fori_loop(0, GPC, group, 0)

    fire(0, bufsets[0])

    def pair(j2, carry):
        c0 = j2 * 2
        fire(c0 + 1, bufsets[1])
        drain(c0, bufsets[0])
        compute(c0, bufsets[0])

        @pl.when(j2 < NCHUNK // 2 - 1)
        def _():
            fire(c0 + 2, bufsets[0])

        drain(c0 + 1, bufsets[1])
        compute(c0 + 1, bufsets[1])
        return carry

    lax.fori_loop(0, NCHUNK // 2, pair, 0)

    pltpu.sync_copy(out_v, out_hbm.at[pl.ds(base, BPW)])


@jax.jit
def _run(heads, rels, tails, cep, cop, rel_all):
    mesh = plsc.VectorSubcoreMesh(core_axis_name="c", subcore_axis_name="s")
    k = functools.partial(
        pl.kernel,
        out_type=jax.ShapeDtypeStruct((B,), jnp.float32),
        mesh=mesh,
        compiler_params=pltpu.CompilerParams(
            needs_layout_passes=False, use_tc_tiling_on_sc=True),
        scratch_types=[
            pltpu.VMEM((BPW,), jnp.int32),
            pltpu.VMEM((BPW,), jnp.int32),
            pltpu.VMEM((BPW,), jnp.int32),
            pltpu.VMEM((BPW,), jnp.int32),
            pltpu.VMEM((BPW,), jnp.int32),
            pltpu.VMEM((CHUNK, 4 * D), jnp.int32),
            pltpu.VMEM((CHUNK, 4 * D), jnp.int32),
            pltpu.VMEM((CHUNK, 4 * D), jnp.int32),
            pltpu.VMEM((CHUNK, 4 * D), jnp.float32),
            pltpu.VMEM((CHUNK, 4 * D), jnp.int32),
            pltpu.VMEM((CHUNK, 4 * D), jnp.int32),
            pltpu.VMEM((CHUNK, 4 * D), jnp.int32),
            pltpu.VMEM((CHUNK, 4 * D), jnp.float32),
            pltpu.VMEM((BPW,), jnp.float32),
            pltpu.SemaphoreType.DMA,
            pltpu.SemaphoreType.DMA,
        ],
    )(_body)
    return k(heads, rels, tails, cep, cop, rel_all)


def kernel(heads, rels, tails, class_embed, class_offset, rel_embed,
           rel_factor, scale_embed, scale_bias):
    cep, cop = _repack(class_embed.T, class_offset.T)
    rel_all = jnp.concatenate(
        [rel_embed, rel_factor, scale_embed, scale_bias], axis=1)
    return _run(heads.astype(jnp.int32), rels.astype(jnp.int32),
                tails.astype(jnp.int32), cep, cop, rel_all)
